# baseline (speedup 1.0000x reference)
import functools
import math

import jax
import jax.numpy as jnp
from jax import lax
from jax.experimental import pallas as pl
from jax.experimental.pallas import tpu as pltpu
from jax.experimental.pallas import tpu_sc as plsc

HEIGHT = 20
WIDTH = 32
D_HALF = 64
N_SPECIAL = 2

TABLE_ROWS = 648
D_MODEL = 2 * D_HALF
GROUP = 160


def _table_body(scale_ref, h_ref, w_ref, out_ref):
    s = scale_ref[0, 0]
    r = lax.broadcasted_iota(jnp.int32, (TABLE_ROWS, WIDTH), 0)
    c = lax.broadcasted_iota(jnp.int32, (TABLE_ROWS, WIDTH), 1)
    a = jnp.maximum(r - N_SPECIAL, 0).astype(jnp.float32)
    wdiv = jnp.float32(WIDTH) / s.astype(jnp.float32)
    off = (s // 2).astype(jnp.float32)
    q = jnp.floor(a / wdiv)
    xi = (q + off).astype(jnp.int32)
    yi = (a - q * wdiv + off).astype(jnp.int32)
    oh_x = (c == xi).astype(jnp.float32)
    oh_y = (c == yi).astype(jnp.float32)
    pe_x = lax.dot(oh_x, h_ref[:, :], preferred_element_type=jnp.float32)
    pe_y = lax.dot(oh_y, w_ref[:, :], preferred_element_type=jnp.float32)
    valid = (r[:, :1] >= N_SPECIAL).astype(jnp.float32)
    out_ref[:, :] = jnp.concatenate([pe_x, pe_y], axis=1) * valid


def _build_table(scale, pos_h_embedding, pos_w_embedding):
    h_pad = jnp.zeros((WIDTH, D_HALF), jnp.float32).at[:HEIGHT].set(pos_h_embedding)
    scale_arr = jnp.asarray(scale, jnp.int32).reshape(1, 1)
    return pl.pallas_call(
        _table_body,
        out_shape=jax.ShapeDtypeStruct((TABLE_ROWS, D_MODEL), jnp.float32),
        in_specs=[
            pl.BlockSpec(memory_space=pltpu.SMEM),
            pl.BlockSpec(memory_space=pltpu.VMEM),
            pl.BlockSpec(memory_space=pltpu.VMEM),
        ],
        out_specs=pl.BlockSpec(memory_space=pltpu.VMEM),
    )(scale_arr, h_pad, pos_w_embedding)


def _sc_gather(table, idx_flat, nw):
    ntok = idx_flat.shape[0]
    tpw = ntok // nw
    rpw = tpw // GROUP

    mesh = plsc.VectorSubcoreMesh(core_axis_name="c", subcore_axis_name="s")

    @functools.partial(
        pl.kernel,
        mesh=mesh,
        out_type=jax.ShapeDtypeStruct((ntok, D_MODEL), jnp.float32),
        compiler_params=pltpu.CompilerParams(needs_layout_passes=False),
        scratch_types=[
            pltpu.VMEM(table.shape, jnp.float32),
            pltpu.VMEM((tpw,), jnp.int32),
            pltpu.VMEM((GROUP, D_MODEL), jnp.float32),
            pltpu.VMEM((GROUP, D_MODEL), jnp.float32),
            pltpu.SemaphoreType.DMA,
            pltpu.SemaphoreType.DMA,
        ],
    )
    def k(table_hbm, idx_hbm, out_hbm, table_v, idx_v, buf_a, buf_b, sem_a, sem_b):
        ncores = jax.lax.axis_size("c")
        wid = lax.axis_index("s") * ncores + lax.axis_index("c")
        pltpu.sync_copy(table_hbm, table_v)
        pltpu.sync_copy(idx_hbm.at[pl.ds(wid * tpw, tpw)], idx_v)
        lane = lax.broadcasted_iota(jnp.int32, (16,), 0)
        cvecs = [lane + 16 * u for u in range(D_MODEL // 16)]

        def fill(j, buf):
            @plsc.parallel_loop(0, GROUP // 4, unroll=2)
            def _(i):
                tok = idx_v[pl.ds(j * GROUP + i // 4 * 16, 16)]
                for q in range(4):
                    t = i * 4 + q
                    tok_s = jnp.take_along_axis(
                        tok, jnp.full((16,), (i % 4) * 4 + q, jnp.int32),
                        axis=0)
                    for u in range(D_MODEL // 16):
                        v = plsc.load_gather(table_v, [tok_s, cvecs[u]])
                        buf[t, pl.ds(u * 16, 16)] = v

        def out_copy(j, buf, sem):
            base = (wid * rpw + j) * GROUP
            return pltpu.make_async_copy(
                buf, out_hbm.at[pl.ds(base, GROUP)], sem)

        def body(g, carry):
            j0 = 2 * g
            j1 = j0 + 1

            @pl.when(g > 0)
            def _():
                out_copy(j0 - 2, buf_a, sem_a).wait()

            fill(j0, buf_a)
            out_copy(j0, buf_a, sem_a).start()

            @pl.when(g > 0)
            def _():
                out_copy(j1 - 2, buf_b, sem_b).wait()

            fill(j1, buf_b)
            out_copy(j1, buf_b, sem_b).start()
            return carry

        lax.fori_loop(0, rpw // 2, body, 0)
        out_copy(rpw - 2, buf_a, sem_a).wait()
        out_copy(rpw - 1, buf_b, sem_b).wait()

    return k(table, idx_flat)


def kernel(tgt_seq, scale, pos_h_embedding, pos_w_embedding):
    b, t = tgt_seq.shape
    table = _build_table(scale, pos_h_embedding, pos_w_embedding)
    info = plsc.get_sparse_core_info()
    nw = info.num_cores * info.num_subcores
    out = _sc_gather(table, tgt_seq.reshape(-1), nw)
    return out.reshape(b, t, D_MODEL)


# speedup vs baseline: 1.0388x; 1.0388x over previous
import functools
import math

import jax
import jax.numpy as jnp
from jax import lax
from jax.experimental import pallas as pl
from jax.experimental.pallas import tpu as pltpu
from jax.experimental.pallas import tpu_sc as plsc

HEIGHT = 20
WIDTH = 32
D_HALF = 64
N_SPECIAL = 2

TABLE_ROWS = 648
D_MODEL = 2 * D_HALF
GROUP = 128


def _table_body(scale_ref, h_ref, w_ref, out_ref):
    s = scale_ref[0, 0]
    r = lax.broadcasted_iota(jnp.int32, (TABLE_ROWS, WIDTH), 0)
    c = lax.broadcasted_iota(jnp.int32, (TABLE_ROWS, WIDTH), 1)
    a = jnp.maximum(r - N_SPECIAL, 0).astype(jnp.float32)
    wdiv = jnp.float32(WIDTH) / s.astype(jnp.float32)
    off = (s // 2).astype(jnp.float32)
    q = jnp.floor(a / wdiv)
    xi = (q + off).astype(jnp.int32)
    yi = (a - q * wdiv + off).astype(jnp.int32)
    oh_x = (c == xi).astype(jnp.float32)
    oh_y = (c == yi).astype(jnp.float32)
    pe_x = lax.dot(oh_x, h_ref[:, :], preferred_element_type=jnp.float32)
    pe_y = lax.dot(oh_y, w_ref[:, :], preferred_element_type=jnp.float32)
    valid = (r[:, :1] >= N_SPECIAL).astype(jnp.float32)
    out_ref[:, :] = jnp.concatenate([pe_x, pe_y], axis=1) * valid


def _build_table(scale, pos_h_embedding, pos_w_embedding):
    h_pad = jnp.zeros((WIDTH, D_HALF), jnp.float32).at[:HEIGHT].set(pos_h_embedding)
    scale_arr = jnp.asarray(scale, jnp.int32).reshape(1, 1)
    return pl.pallas_call(
        _table_body,
        out_shape=jax.ShapeDtypeStruct((TABLE_ROWS, D_MODEL), jnp.float32),
        in_specs=[
            pl.BlockSpec(memory_space=pltpu.SMEM),
            pl.BlockSpec(memory_space=pltpu.VMEM),
            pl.BlockSpec(memory_space=pltpu.VMEM),
        ],
        out_specs=pl.BlockSpec(memory_space=pltpu.VMEM),
    )(scale_arr, h_pad, pos_w_embedding)


def _sc_gather(table, tgt2d, nw):
    nb, tcols = tgt2d.shape
    ntok = nb * tcols
    bpw = nb // nw
    tpw = ntok // nw
    rpw = tpw // GROUP
    nfull = tcols // 16

    mesh = plsc.VectorSubcoreMesh(core_axis_name="c", subcore_axis_name="s")

    @functools.partial(
        pl.kernel,
        mesh=mesh,
        out_type=jax.ShapeDtypeStruct((ntok, D_MODEL), jnp.float32),
        compiler_params=pltpu.CompilerParams(needs_layout_passes=False),
        scratch_types=[
            pltpu.VMEM(table.shape, jnp.float32),
            pltpu.VMEM((bpw, tcols), jnp.int32),
            pltpu.VMEM((tpw,), jnp.int32),
            pltpu.VMEM((GROUP, D_MODEL), jnp.float32),
            pltpu.VMEM((GROUP, D_MODEL), jnp.float32),
            pltpu.SemaphoreType.DMA,
            pltpu.SemaphoreType.DMA,
        ],
    )
    def k(table_hbm, idx_hbm, out_hbm, table_v, idx_raw, idx_v,
          buf_a, buf_b, sem_a, sem_b):
        ncores = jax.lax.axis_size("c")
        wid = lax.axis_index("s") * ncores + lax.axis_index("c")
        pltpu.sync_copy(table_hbm, table_v)
        pltpu.sync_copy(idx_hbm.at[pl.ds(wid * bpw, bpw)], idx_raw)

        @plsc.parallel_loop(0, bpw, unroll=1)
        def _(r):
            for u in range(nfull):
                idx_v[pl.ds(r * tcols + u * 16, 16)] = (
                    idx_raw[r, pl.ds(u * 16, 16)])
            if tcols % 16:
                idx_v[pl.ds(r * tcols + tcols - 16, 16)] = (
                    idx_raw[r, pl.ds(tcols - 16, 16)])

        lane = lax.broadcasted_iota(jnp.int32, (16,), 0)
        cvecs = [lane + 16 * u for u in range(D_MODEL // 16)]

        def fill(j, buf):
            @plsc.parallel_loop(0, GROUP, unroll=4)
            def _(t):
                tv = t // 16 * 16
                tok = idx_v[pl.ds(j * GROUP + tv, 16)]
                tok_s = jnp.take_along_axis(
                    tok, jnp.full((16,), t % 16, jnp.int32), axis=0)
                for u in range(D_MODEL // 16):
                    v = plsc.load_gather(table_v, [tok_s, cvecs[u]])
                    buf[t, pl.ds(u * 16, 16)] = v

        def out_copy(j, buf, sem):
            base = (wid * rpw + j) * GROUP
            return pltpu.make_async_copy(
                buf, out_hbm.at[pl.ds(base, GROUP)], sem)

        def body(g, carry):
            j0 = 2 * g
            j1 = j0 + 1

            @pl.when(g > 0)
            def _():
                out_copy(j0 - 2, buf_a, sem_a).wait()

            fill(j0, buf_a)
            out_copy(j0, buf_a, sem_a).start()

            @pl.when(g > 0)
            def _():
                out_copy(j1 - 2, buf_b, sem_b).wait()

            fill(j1, buf_b)
            out_copy(j1, buf_b, sem_b).start()
            return carry

        lax.fori_loop(0, rpw // 2, body, 0)
        out_copy(rpw - 2, buf_a, sem_a).wait()
        out_copy(rpw - 1, buf_b, sem_b).wait()

    return k(table, tgt2d)


def kernel(tgt_seq, scale, pos_h_embedding, pos_w_embedding):
    b, t = tgt_seq.shape
    table = _build_table(scale, pos_h_embedding, pos_w_embedding)
    info = plsc.get_sparse_core_info()
    nw = info.num_cores * info.num_subcores
    out = _sc_gather(table, tgt_seq, nw)
    return out.reshape(b, t, D_MODEL)


# speedup vs baseline: 1.0716x; 1.0315x over previous
import functools
import math

import jax
import jax.numpy as jnp
from jax import lax
from jax.experimental import pallas as pl
from jax.experimental.pallas import tpu as pltpu
from jax.experimental.pallas import tpu_sc as plsc

HEIGHT = 20
WIDTH = 32
D_HALF = 64
N_SPECIAL = 2

NVALS = 642
MAPN = 648
D_MODEL = 2 * D_HALF
GROUP = 128


def _index_maps(scale):
    v = jnp.arange(MAPN, dtype=jnp.int32)
    a = jnp.maximum(v - N_SPECIAL, 0)
    wdiv = WIDTH / scale
    q = jnp.floor(a / wdiv)
    xi = (q + scale // 2).astype(jnp.int32)
    yi = (a - q * wdiv + scale // 2).astype(jnp.int32)
    valid = v >= N_SPECIAL
    xmap = jnp.where(valid, jnp.clip(xi, 0, HEIGHT - 1), HEIGHT)
    ymap = jnp.where(valid, jnp.clip(yi, 0, WIDTH - 1), WIDTH)
    return xmap.astype(jnp.int32), ymap.astype(jnp.int32)


def _sc_gather(h_ext, w_ext, xmap, ymap, tgt2d, nw):
    nb, tcols = tgt2d.shape
    ntok = nb * tcols
    bpw = nb // nw
    tpw = ntok // nw
    rpw = tpw // GROUP
    nfull = tcols // 16

    mesh = plsc.VectorSubcoreMesh(core_axis_name="c", subcore_axis_name="s")

    @functools.partial(
        pl.kernel,
        mesh=mesh,
        out_type=jax.ShapeDtypeStruct((ntok, D_MODEL), jnp.float32),
        compiler_params=pltpu.CompilerParams(needs_layout_passes=False),
        scratch_types=[
            pltpu.VMEM(h_ext.shape, jnp.float32),
            pltpu.VMEM(w_ext.shape, jnp.float32),
            pltpu.VMEM((MAPN,), jnp.int32),
            pltpu.VMEM((MAPN,), jnp.int32),
            pltpu.VMEM((bpw, tcols), jnp.int32),
            pltpu.VMEM((tpw,), jnp.int32),
            pltpu.VMEM((GROUP, D_MODEL), jnp.float32),
            pltpu.VMEM((GROUP, D_MODEL), jnp.float32),
            pltpu.SemaphoreType.DMA,
            pltpu.SemaphoreType.DMA,
        ],
    )
    def k(h_hbm, w_hbm, xmap_hbm, ymap_hbm, idx_hbm, out_hbm,
          h_v, w_v, xmap_v, ymap_v, idx_raw, idx_v,
          buf_a, buf_b, sem_a, sem_b):
        ncores = jax.lax.axis_size("c")
        wid = lax.axis_index("s") * ncores + lax.axis_index("c")
        pltpu.sync_copy(h_hbm, h_v)
        pltpu.sync_copy(w_hbm, w_v)
        pltpu.sync_copy(xmap_hbm, xmap_v)
        pltpu.sync_copy(ymap_hbm, ymap_v)
        pltpu.sync_copy(idx_hbm.at[pl.ds(wid * bpw, bpw)], idx_raw)

        @plsc.parallel_loop(0, bpw, unroll=1)
        def _(r):
            for u in range(nfull):
                idx_v[pl.ds(r * tcols + u * 16, 16)] = (
                    idx_raw[r, pl.ds(u * 16, 16)])
            if tcols % 16:
                idx_v[pl.ds(r * tcols + tcols - 16, 16)] = (
                    idx_raw[r, pl.ds(tcols - 16, 16)])

        lane = lax.broadcasted_iota(jnp.int32, (16,), 0)
        cvecs = [lane + 16 * u for u in range(D_HALF // 16)]

        def fill(j, buf):
            @plsc.parallel_loop(0, GROUP, unroll=4)
            def _(t):
                tv = t // 16 * 16
                tok = idx_v[pl.ds(j * GROUP + tv, 16)]
                xi = plsc.load_gather(xmap_v, [tok])
                yi = plsc.load_gather(ymap_v, [tok])
                pidx = jnp.full((16,), t % 16, jnp.int32)
                xs = jnp.take_along_axis(xi, pidx, axis=0)
                ys = jnp.take_along_axis(yi, pidx, axis=0)
                for u in range(D_HALF // 16):
                    buf[t, pl.ds(u * 16, 16)] = (
                        plsc.load_gather(h_v, [xs, cvecs[u]]))
                for u in range(D_HALF // 16):
                    buf[t, pl.ds(D_HALF + u * 16, 16)] = (
                        plsc.load_gather(w_v, [ys, cvecs[u]]))

        def out_copy(j, buf, sem):
            base = (wid * rpw + j) * GROUP
            return pltpu.make_async_copy(
                buf, out_hbm.at[pl.ds(base, GROUP)], sem)

        def body(g, carry):
            j0 = 2 * g
            j1 = j0 + 1

            @pl.when(g > 0)
            def _():
                out_copy(j0 - 2, buf_a, sem_a).wait()

            fill(j0, buf_a)
            out_copy(j0, buf_a, sem_a).start()

            @pl.when(g > 0)
            def _():
                out_copy(j1 - 2, buf_b, sem_b).wait()

            fill(j1, buf_b)
            out_copy(j1, buf_b, sem_b).start()
            return carry

        lax.fori_loop(0, rpw // 2, body, 0)
        out_copy(rpw - 2, buf_a, sem_a).wait()
        out_copy(rpw - 1, buf_b, sem_b).wait()

    return k(h_ext, w_ext, xmap, ymap, tgt2d)


def kernel(tgt_seq, scale, pos_h_embedding, pos_w_embedding):
    b, t = tgt_seq.shape
    h_ext = jnp.zeros((HEIGHT + 1, D_HALF), jnp.float32).at[:HEIGHT].set(
        pos_h_embedding)
    w_ext = jnp.zeros((WIDTH + 1, D_HALF), jnp.float32).at[:WIDTH].set(
        pos_w_embedding)
    xmap, ymap = _index_maps(scale)
    info = plsc.get_sparse_core_info()
    nw = info.num_cores * info.num_subcores
    out = _sc_gather(h_ext, w_ext, xmap, ymap, tgt_seq, nw)
    return out.reshape(b, t, D_MODEL)
